# baseline (speedup 1.0000x reference)
import functools

import jax
import jax.numpy as jnp
import numpy as np
from jax.experimental import pallas as pl

_K = 64
_W = 64
_INT_MIN = np.int32(-(2 ** 31))
_TOP_MASK = np.int32(0x7FFFFFFF)


def _sortable(x):
    xi = jax.lax.bitcast_convert_type(x, jnp.int32)
    return xi ^ (jax.lax.shift_right_arithmetic(xi, 31) & _TOP_MASK)


def _radix_kth(s, k, axes):
    def body(i, p):
        bit = jax.lax.shift_left(np.int32(1), (31 - i).astype(jnp.int32))
        cand = p | bit
        thr = cand ^ _INT_MIN
        cnt = jnp.sum((s >= thr).astype(jnp.int32), axis=axes, keepdims=True)
        return jnp.where(cnt >= k, cand, p)

    shape = tuple(1 if d in axes else n for d, n in enumerate(s.shape))
    p = jax.lax.fori_loop(0, 32, body, jnp.zeros(shape, jnp.int32), unroll=True)
    return p ^ _INT_MIN


def _topk_mask_kernel(x_ref, o_ref, *, k):
    x = x_ref[...]
    r, w, g = x.shape
    s = _sortable(x)

    sg = jnp.max(s, axis=1)
    vg = _radix_kth(sg, k, axes=(1,))

    gt = sg > vg
    eq = sg == vg
    gtf = gt.astype(jnp.float32)
    eqf = eq.astype(jnp.float32)
    ia = jax.lax.broadcasted_iota(jnp.int32, (g, g), 0)
    ib = jax.lax.broadcasted_iota(jnp.int32, (g, g), 1)
    tri = (ia <= ib).astype(jnp.float32)
    cum_gt = jnp.dot(gtf, tri, preferred_element_type=jnp.float32)
    cum_eq = jnp.dot(eqf, tri, preferred_element_type=jnp.float32)
    cgt = jnp.sum(gtf, axis=1, keepdims=True)
    rgt = cum_gt - gtf
    req = cum_eq - eqf + cgt
    rank = jnp.where(gt, rgt, jnp.where(eq, req, np.float32(1e9)))

    slots = jax.lax.broadcasted_iota(jnp.int32, (1, k, 1), 1).astype(jnp.float32)
    sel = (rank[:, None, :] == slots).astype(jnp.float32)
    compact = jax.lax.dot_general(
        sel, x, (((2,), (2,)), ((0,), (0,))),
        precision=jax.lax.Precision.HIGHEST,
        preferred_element_type=jnp.float32)

    v = _radix_kth(_sortable(compact), k, axes=(1, 2))

    o_ref[...] = jnp.where(s >= v, jnp.maximum(x, 0.0), 0.0)


def _topk_mask_3d(x3, k, rows_per_block):
    n_rows, w, g = x3.shape
    body = functools.partial(_topk_mask_kernel, k=k)
    return pl.pallas_call(
        body,
        grid=(n_rows // rows_per_block,),
        in_specs=[pl.BlockSpec((rows_per_block, w, g), lambda i: (i, 0, 0))],
        out_specs=pl.BlockSpec((rows_per_block, w, g), lambda i: (i, 0, 0)),
        out_shape=jax.ShapeDtypeStruct((n_rows, w, g), x3.dtype),
    )(x3)


def kernel(features):
    b, l, d = features.shape
    x3 = features.reshape(b * l, _W, d // _W)
    out = _topk_mask_3d(x3, _K, 8)
    return out.reshape(b, l, d)


# speedup vs baseline: 1.6630x; 1.6630x over previous
import functools

import jax
import jax.numpy as jnp
import numpy as np
from jax.experimental import pallas as pl

_K = 64
_W = 64
_INT_MIN = np.int32(-(2 ** 31))
_TOP_MASK = np.int32(0x7FFFFFFF)


def _sortable(x):
    xi = jax.lax.bitcast_convert_type(x, jnp.int32)
    return xi ^ (jax.lax.shift_right_arithmetic(xi, 31) & _TOP_MASK)


def _radix_kth(s, k, axes):
    def body(i, p):
        bit = jax.lax.shift_left(np.int32(1), (31 - i).astype(jnp.int32))
        cand = p | bit
        thr = cand ^ _INT_MIN
        cnt = jnp.sum((s >= thr).astype(jnp.int32), axis=axes, keepdims=True)
        return jnp.where(cnt >= k, cand, p)

    shape = tuple(1 if d in axes else n for d, n in enumerate(s.shape))
    p = jax.lax.fori_loop(0, 32, body, jnp.zeros(shape, jnp.int32), unroll=True)
    return p ^ _INT_MIN


def _topk_mask_kernel(x_ref, o_ref, *, k):
    x = x_ref[...]
    r, w, g = x.shape
    s = _sortable(x)

    sg = jnp.max(s, axis=1)
    vg = _radix_kth(sg, k, axes=(1,))

    gt = sg > vg
    eq = sg == vg
    gtf = gt.astype(jnp.float32)
    eqf = eq.astype(jnp.float32)
    ia = jax.lax.broadcasted_iota(jnp.int32, (g, g), 0)
    ib = jax.lax.broadcasted_iota(jnp.int32, (g, g), 1)
    tri = (ia <= ib).astype(jnp.float32)
    cum_gt = jnp.dot(gtf, tri, preferred_element_type=jnp.float32)
    cum_eq = jnp.dot(eqf, tri, preferred_element_type=jnp.float32)
    cgt = jnp.sum(gtf, axis=1, keepdims=True)
    rgt = cum_gt - gtf
    req = cum_eq - eqf + cgt
    rank = jnp.where(gt, rgt, jnp.where(eq, req, np.float32(1e9)))

    slots = jax.lax.broadcasted_iota(jnp.int32, (1, k, 1), 1).astype(jnp.float32)
    sel = (rank[:, None, :] == slots).astype(jnp.float32)
    compact = jax.lax.dot_general(
        sel, x, (((2,), (2,)), ((0,), (0,))),
        precision=jax.lax.Precision.HIGHEST,
        preferred_element_type=jnp.float32)

    v = _radix_kth(_sortable(compact), k, axes=(1, 2))

    o_ref[...] = jnp.where(s >= v, jnp.maximum(x, 0.0), 0.0)


def _topk_mask_3d(x3, k, rows_per_block):
    n_rows, w, g = x3.shape
    body = functools.partial(_topk_mask_kernel, k=k)
    return pl.pallas_call(
        body,
        grid=(n_rows // rows_per_block,),
        in_specs=[pl.BlockSpec((rows_per_block, w, g), lambda i: (i, 0, 0))],
        out_specs=pl.BlockSpec((rows_per_block, w, g), lambda i: (i, 0, 0)),
        out_shape=jax.ShapeDtypeStruct((n_rows, w, g), x3.dtype),
    )(x3)


def kernel(features):
    b, l, d = features.shape
    x3 = features.reshape(b * l, _W, d // _W)
    out = _topk_mask_3d(x3, _K, 32)
    return out.reshape(b, l, d)
